# baseline (speedup 1.0000x reference)
import functools

import jax
import jax.numpy as jnp
from jax import lax
from jax.experimental import pallas as pl
from jax.experimental.pallas import tpu as pltpu
from jax.experimental.pallas import tpu_sc as plsc

NUM_ROWS = 1000000
DIM = 32
QMAX = 255.0
B_TOTAL = 4096 * 50
NUM_WORKERS = 32
B_PER_W = B_TOTAL // NUM_WORKERS
CHUNK = 1600
N_CHUNKS = B_PER_W // CHUNK

_MM_BLOCK = 8000
_TX_BLOCK = 6400

def _minmax_body(w_ref, scale_ref, zp_ref, mn_ref, mx_ref):
    i = pl.program_id(0)
    blk = w_ref[...]
    bmn = jnp.min(blk, axis=0, keepdims=True)
    bmx = jnp.max(blk, axis=0, keepdims=True)

    @pl.when(i == 0)
    def _():
        mn_ref[...] = bmn
        mx_ref[...] = bmx

    @pl.when(i > 0)
    def _():
        mn_ref[...] = jnp.minimum(mn_ref[...], bmn)
        mx_ref[...] = jnp.maximum(mx_ref[...], bmx)

    @pl.when(i == pl.num_programs(0) - 1)
    def _():
        mn = jnp.minimum(jnp.min(mn_ref[...]), 0.0)
        mx = jnp.maximum(jnp.max(mx_ref[...]), 0.0)
        s = jnp.maximum((mx - mn) / QMAX, 1e-8)
        zp = jnp.clip(jnp.round(-mn / s), 0.0, QMAX)
        scale_ref[0, 0] = s
        zp_ref[0, 0] = zp


def _quant_params_pallas(weight):
    return pl.pallas_call(
        _minmax_body,
        grid=(NUM_ROWS // _MM_BLOCK,),
        in_specs=[pl.BlockSpec((_MM_BLOCK, DIM), lambda i: (i, 0))],
        out_specs=[pl.BlockSpec(memory_space=pltpu.SMEM),
                   pl.BlockSpec(memory_space=pltpu.SMEM)],
        out_shape=[jax.ShapeDtypeStruct((1, 1), jnp.float32),
                   jax.ShapeDtypeStruct((1, 1), jnp.float32)],
        scratch_shapes=[pltpu.VMEM((1, DIM), jnp.float32),
                        pltpu.VMEM((1, DIM), jnp.float32)],
    )(weight)


@functools.cache
def _gather_kernel():
    @functools.partial(
        pl.kernel,
        mesh=plsc.VectorSubcoreMesh(core_axis_name="c", subcore_axis_name="s"),
        out_type=[jax.ShapeDtypeStruct((B_TOTAL, DIM), jnp.float32),
                  jax.ShapeDtypeStruct((B_TOTAL, DIM), jnp.uint8)],
        scratch_types=[pltpu.VMEM((CHUNK,), jnp.int32),
                       pltpu.VMEM((CHUNK, DIM), jnp.float32),
                       pltpu.VMEM((CHUNK, DIM), jnp.uint8),
                       pltpu.SemaphoreType.DMA,
                       pltpu.SemaphoreType.DMA],
        compiler_params=pltpu.CompilerParams(use_tc_tiling_on_sc=False),
    )
    def _gather_sc(idx_hbm, w_hbm, m_hbm, out_w, out_m,
                   idx_v, wbuf, mbuf, sem_w, sem_m):
        wid = lax.axis_index("s") * 2 + lax.axis_index("c")
        base = wid * B_PER_W
        for c in range(N_CHUNKS):
            off = base + c * CHUNK
            pltpu.sync_copy(idx_hbm.at[pl.ds(off, CHUNK)], idx_v)
            cp_w = pltpu.async_copy(w_hbm.at[idx_v], wbuf, sem_w)
            cp_m = pltpu.async_copy(m_hbm.at[idx_v], mbuf, sem_m)
            cp_w.wait()
            cp_m.wait()
            pltpu.sync_copy(wbuf, out_w.at[pl.ds(off, CHUNK)])
            pltpu.sync_copy(mbuf, out_m.at[pl.ds(off, CHUNK)])

    return _gather_sc


def _transform_body(scale_ref, zp_ref, w_ref, m_ref, o_ref):
    s = scale_ref[0, 0]
    zp = zp_ref[0, 0]
    w = w_ref[...]
    keep = m_ref[...] == 0
    q = jnp.clip(jnp.round(w / s + zp), 0.0, QMAX)
    wq = (q - zp) * s
    clamped = jnp.clip(w, -s * zp, s * (QMAX - zp))
    o_ref[...] = clamped + jnp.where(keep, wq - w, 0.0)


def _transform_pallas(scale, zp, gw, gm):
    return pl.pallas_call(
        _transform_body,
        grid=(B_TOTAL // _TX_BLOCK,),
        in_specs=[pl.BlockSpec(memory_space=pltpu.SMEM),
                  pl.BlockSpec(memory_space=pltpu.SMEM),
                  pl.BlockSpec((_TX_BLOCK, DIM), lambda i: (i, 0)),
                  pl.BlockSpec((_TX_BLOCK, DIM), lambda i: (i, 0))],
        out_specs=pl.BlockSpec((_TX_BLOCK, DIM), lambda i: (i, 0)),
        out_shape=jax.ShapeDtypeStruct((B_TOTAL, DIM), jnp.float32),
    )(scale, zp, gw, gm)


def kernel(input, weight, mask):
    idx = input.reshape(-1)
    m8 = mask.astype(jnp.uint8)
    scale, zp = _quant_params_pallas(weight)
    gw, gm = _gather_kernel()(idx, weight, m8)
    out = _transform_pallas(scale, zp, gw, gm)
    return out.reshape(input.shape + (DIM,))


# speedup vs baseline: 1.6545x; 1.6545x over previous
import functools

import jax
import jax.numpy as jnp
from jax import lax
from jax.experimental import pallas as pl
from jax.experimental.pallas import tpu as pltpu
from jax.experimental.pallas import tpu_sc as plsc

NUM_ROWS = 1000000
DIM = 32
QMAX = 255.0
B_TOTAL = 4096 * 50
NUM_WORKERS = 32
B_PER_W = B_TOTAL // NUM_WORKERS
CHUNK = 1600
N_CHUNKS = B_PER_W // CHUNK

_MM_BLOCK = 8192
_TX_BLOCK = 8192


def _minmax_body(wt_ref, scale_ref, zp_ref, mn_ref, mx_ref):
    i = pl.program_id(0)
    blk = wt_ref[...]
    col = i * _MM_BLOCK + lax.broadcasted_iota(jnp.int32, blk.shape, 1)
    valid = col < NUM_ROWS
    bmn = jnp.min(jnp.where(valid, blk, jnp.inf))
    bmx = jnp.max(jnp.where(valid, blk, -jnp.inf))

    @pl.when(i == 0)
    def _():
        mn_ref[0, 0] = bmn
        mx_ref[0, 0] = bmx

    @pl.when(i > 0)
    def _():
        mn_ref[0, 0] = jnp.minimum(mn_ref[0, 0], bmn)
        mx_ref[0, 0] = jnp.maximum(mx_ref[0, 0], bmx)

    @pl.when(i == pl.num_programs(0) - 1)
    def _():
        mn = jnp.minimum(mn_ref[0, 0], 0.0)
        mx = jnp.maximum(mx_ref[0, 0], 0.0)
        s = jnp.maximum((mx - mn) / QMAX, 1e-8)
        zp = jnp.clip(jnp.round(-mn / s), 0.0, QMAX)
        scale_ref[0, 0] = s
        zp_ref[0, 0] = zp


def _quant_params_pallas(wt):
    return pl.pallas_call(
        _minmax_body,
        grid=(pl.cdiv(NUM_ROWS, _MM_BLOCK),),
        in_specs=[pl.BlockSpec((DIM, _MM_BLOCK), lambda i: (0, i))],
        out_specs=[pl.BlockSpec(memory_space=pltpu.SMEM),
                   pl.BlockSpec(memory_space=pltpu.SMEM)],
        out_shape=[jax.ShapeDtypeStruct((1, 1), jnp.float32),
                   jax.ShapeDtypeStruct((1, 1), jnp.float32)],
        scratch_shapes=[pltpu.SMEM((1, 1), jnp.float32),
                        pltpu.SMEM((1, 1), jnp.float32)],
    )(wt)


def _transform_body(scale_ref, zp_ref, wt_ref, mt_ref, o_ref):
    s = scale_ref[0, 0]
    zp = zp_ref[0, 0]
    w = wt_ref[...]
    m = mt_ref[...]
    q = jnp.clip(jnp.round(w / s + zp), 0.0, QMAX)
    wq = (q - zp) * s
    o = jnp.clip(w, -s * zp, s * (QMAX - zp)) + jnp.where(m, 0.0, wq - w)
    o_ref[...] = o.T


def _transform_pallas(scale, zp, wt, mt):
    return pl.pallas_call(
        _transform_body,
        grid=(pl.cdiv(NUM_ROWS, _TX_BLOCK),),
        in_specs=[pl.BlockSpec(memory_space=pltpu.SMEM),
                  pl.BlockSpec(memory_space=pltpu.SMEM),
                  pl.BlockSpec((DIM, _TX_BLOCK), lambda i: (0, i)),
                  pl.BlockSpec((DIM, _TX_BLOCK), lambda i: (0, i))],
        out_specs=pl.BlockSpec((_TX_BLOCK, DIM), lambda i: (i, 0)),
        out_shape=jax.ShapeDtypeStruct((NUM_ROWS, DIM), jnp.float32),
    )(scale, zp, wt, mt)


@functools.cache
def _gather_kernel():
    @functools.partial(
        pl.kernel,
        mesh=plsc.VectorSubcoreMesh(core_axis_name="c", subcore_axis_name="s"),
        out_type=jax.ShapeDtypeStruct((B_TOTAL, DIM), jnp.float32),
        scratch_types=[pltpu.VMEM((CHUNK,), jnp.int32),
                       pltpu.VMEM((CHUNK, DIM), jnp.float32),
                       pltpu.SemaphoreType.DMA],
        compiler_params=pltpu.CompilerParams(use_tc_tiling_on_sc=False),
    )
    def _gather_sc(idx_hbm, w_hbm, out_w, idx_v, wbuf, sem_w):
        wid = lax.axis_index("s") * 2 + lax.axis_index("c")
        base = wid * B_PER_W
        for c in range(N_CHUNKS):
            off = base + c * CHUNK
            pltpu.sync_copy(idx_hbm.at[pl.ds(off, CHUNK)], idx_v)
            pltpu.async_copy(w_hbm.at[idx_v], wbuf, sem_w).wait()
            pltpu.sync_copy(wbuf, out_w.at[pl.ds(off, CHUNK)])

    return _gather_sc


def kernel(input, weight, mask):
    wt = weight.T
    mt = mask.T
    scale, zp = _quant_params_pallas(wt)
    wprime = _transform_pallas(scale, zp, wt, mt)
    idx = input.reshape(-1)
    out = _gather_kernel()(idx, wprime)
    return out.reshape(input.shape + (DIM,))
